# Initial kernel scaffold; baseline (speedup 1.0000x reference)
import jax
import jax.numpy as jnp
from jax.experimental import pallas as pl


def kernel(artist, title, numerical_data, emb_artist, emb_title, W1, b1, W2, b2, W3, b3):
    raise NotImplementedError("write your pallas kernel here")


# speedup vs baseline: 2.1124x; 2.1124x over previous
import functools

import jax
import jax.numpy as jnp
from jax import lax
from jax.experimental import pallas as pl
from jax.experimental.pallas import tpu as pltpu
from jax.experimental.pallas import tpu_sc as plsc

B = 16384
L = 20
DA = 64
DT = 64

NC = 2
NS = 16
NW = NC * NS
RPW = B // NW
CB = 4
IPC = CB * L
NCHUNK = RPW // CB


def _sc_pool_body(artist_ref, title_ref, emb_a_ref, emb_t_ref, out_ref,
                  idx_a, idx_t, rows_a, rows_t, out_v, sem_a, sem_t):
    wid = lax.axis_index("s") * NC + lax.axis_index("c")
    pltpu.sync_copy(artist_ref.at[wid], idx_a)
    pltpu.sync_copy(title_ref.at[wid], idx_t)

    inv_l = jnp.float32(1.0 / L)

    def step(j, carry):
        cp_a = pltpu.async_copy(emb_a_ref.at[idx_a.at[j]], rows_a, sem_a)
        cp_t = pltpu.async_copy(emb_t_ref.at[idx_t.at[j]], rows_t, sem_t)
        cp_a.wait()
        cp_t.wait()
        for r in range(CB):
            row = j * CB + r
            for k in range(DA // 16):
                sl = pl.ds(k * 16, 16)
                acc = rows_a[L * r, sl]
                for q in range(1, L):
                    acc = acc + rows_a[L * r + q, sl]
                out_v[row, sl] = acc * inv_l
            for k in range(DT // 16):
                sl = pl.ds(k * 16, 16)
                acc = rows_t[L * r, sl]
                for q in range(1, L):
                    acc = acc + rows_t[L * r + q, sl]
                out_v[row, pl.ds(DA + k * 16, 16)] = acc * inv_l
        return carry

    lax.fori_loop(0, NCHUNK, step, 0)
    pltpu.sync_copy(out_v, out_ref.at[pl.ds(wid * RPW, RPW)])


@functools.partial(jax.jit, static_argnums=())
def _sc_pool(artist_r, title_r, emb_artist, emb_title):
    mesh = plsc.VectorSubcoreMesh(core_axis_name="c", subcore_axis_name="s",
                                  num_cores=NC, num_subcores=NS)
    return pl.kernel(
        _sc_pool_body,
        out_type=jax.ShapeDtypeStruct((B, DA + DT), jnp.float32),
        mesh=mesh,
        scratch_types=[
            pltpu.VMEM((NCHUNK, IPC), jnp.int32),
            pltpu.VMEM((NCHUNK, IPC), jnp.int32),
            pltpu.VMEM((IPC, DA), jnp.float32),
            pltpu.VMEM((IPC, DT), jnp.float32),
            pltpu.VMEM((RPW, DA + DT), jnp.float32),
            pltpu.SemaphoreType.DMA,
            pltpu.SemaphoreType.DMA,
        ],
        compiler_params=pltpu.CompilerParams(use_tc_tiling_on_sc=False),
    )(artist_r, title_r, emb_artist, emb_title)


def _mlp_body(x_ref, num_ref, w1e_ref, w1n_ref, b1_ref, w2_ref, b2_ref,
              w3t_ref, b3_ref, out_ref):
    x = x_ref[...]
    h1 = jnp.dot(x, w1e_ref[...], preferred_element_type=jnp.float32)
    num = num_ref[...]
    h1 = h1 + num[:, 0:1] * w1n_ref[0:1, :] + num[:, 1:2] * w1n_ref[1:2, :]
    h1 = jnp.maximum(h1 + b1_ref[...], 0.0)
    h2 = jnp.dot(h1, w2_ref[...], preferred_element_type=jnp.float32)
    h2 = jnp.maximum(h2 + b2_ref[...], 0.0)
    out_ref[...] = jnp.sum(h2 * w3t_ref[...], axis=1, keepdims=True) + b3_ref[...]


def _mlp(pooled, num, w1e, w1n, b1, w2, b2, w3t, b3):
    bb = 2048
    grid = (B // bb,)
    return pl.pallas_call(
        _mlp_body,
        grid=grid,
        in_specs=[
            pl.BlockSpec((bb, DA + DT), lambda i: (i, 0)),
            pl.BlockSpec((bb, 2), lambda i: (i, 0)),
            pl.BlockSpec((DA + DT, 128), lambda i: (0, 0)),
            pl.BlockSpec((2, 128), lambda i: (0, 0)),
            pl.BlockSpec((1, 128), lambda i: (0, 0)),
            pl.BlockSpec((128, 64), lambda i: (0, 0)),
            pl.BlockSpec((1, 64), lambda i: (0, 0)),
            pl.BlockSpec((1, 64), lambda i: (0, 0)),
            pl.BlockSpec((1, 1), lambda i: (0, 0)),
        ],
        out_specs=pl.BlockSpec((bb, 1), lambda i: (i, 0)),
        out_shape=jax.ShapeDtypeStruct((B, 1), jnp.float32),
    )(pooled, num, w1e, w1n, b1, w2, b2, w3t, b3)


def kernel(artist, title, numerical_data, emb_artist, emb_title,
           W1, b1, W2, b2, W3, b3):
    artist_r = artist.astype(jnp.int32).reshape(NW, NCHUNK, IPC)
    title_r = title.astype(jnp.int32).reshape(NW, NCHUNK, IPC)
    pooled = _sc_pool(artist_r, title_r, emb_artist, emb_title)
    w1e = W1[: DA + DT]
    w1n = W1[DA + DT:]
    return _mlp(pooled, numerical_data, w1e, w1n, b1.reshape(1, 128),
                W2, b2.reshape(1, 64), W3.reshape(1, 64), b3.reshape(1, 1))
